# baseline (speedup 1.0000x reference)
import functools

import jax
import jax.numpy as jnp
from jax import lax
from jax.experimental import pallas as pl
from jax.experimental.pallas import tpu as pltpu
from jax.experimental.pallas import tpu_sc as plsc

N = 10000
E = 320000
F_IN = 128
EDIM = 4

_K = 64
_NCHUNK = E // _K
_NW = 32
_MAXCPW = (_NCHUNK + _NW - 1) // _NW
_NPAD = 10112
_RPT = _NPAD // 16
_W = 136


def _sc_layer(tab, ald, ei, ale, H):
    C = 128 // H
    mesh = plsc.VectorSubcoreMesh(core_axis_name="c", subcore_axis_name="s")

    def body(tab_hbm, ald_hbm, ei_hbm, ale_hbm, out_hbm,
             acc, eib0, eib1, aleb0, aleb1, gbuf0, gbuf1, adbuf0, adbuf1,
             mb0, mb1, ixb0, ixb1, exb,
             semi0, semi1, semg0, semg1, sems0, sems1):
        cid = lax.axis_index("c")
        sid = lax.axis_index("s")
        wid = sid * 2 + cid
        eib = [eib0, eib1]
        aleb = [aleb0, aleb1]
        gbuf = [gbuf0, gbuf1]
        adbuf = [adbuf0, adbuf1]
        mb = [mb0, mb1]
        ixb = [ixb0, ixb1]
        semi = [semi0, semi1]
        semg = [semg0, semg1]
        sems = [sems0, sems1]

        zz = jnp.zeros((16,), jnp.float32)
        for mbuf in (mb0, mb1):
            for e in range(_K):
                for cg in range(8):
                    mbuf[e, pl.ds(cg * 16, 16)] = zz
                mbuf[e, pl.ds(_W - 16, 16)] = zz
        row0 = sid * _RPT
        nz = _RPT // _K
        for j in range(nz + 1):
            r = _K if j < nz else (_RPT - nz * _K)
            pltpu.sync_copy(mb0.at[pl.ds(0, r)],
                            acc.at[pl.ds(row0 + j * _K, r)])
        plsc.subcore_barrier()

        iota = lax.iota(jnp.int32, 16)
        n_i = (_NCHUNK - wid + _NW - 1) // _NW

        def base_of(ch):
            return (wid + ch * _NW) * _K

        def issue_idx(ch, b):
            pltpu.async_copy(ei_hbm.at[:, pl.ds(base_of(ch), _K)],
                             eib[b], semi[b])
            pltpu.async_copy(ale_hbm.at[:, pl.ds(base_of(ch), _K)],
                             aleb[b], semi[b])

        def wait_idx(b):
            pltpu.make_async_copy(ei_hbm.at[:, pl.ds(0, _K)],
                                  eib[b], semi[b]).wait()
            pltpu.make_async_copy(ale_hbm.at[:, pl.ds(0, _K)],
                                  aleb[b], semi[b]).wait()

        def issue_gather(b):
            pltpu.async_copy(tab_hbm.at[eib[b].at[0]], gbuf[b], semg[b])
            pltpu.async_copy(ald_hbm.at[eib[b].at[1]], adbuf[b], semg[b])

        def wait_gather(b):
            pltpu.make_async_copy(tab_hbm.at[eib[b].at[0]],
                                  gbuf[b], semg[b]).wait()
            pltpu.make_async_copy(ald_hbm.at[eib[b].at[1]],
                                  adbuf[b], semg[b]).wait()

        pltpu.sync_copy(ei_hbm.at[:, pl.ds(base_of(0), _K)], eib0)
        pltpu.sync_copy(ale_hbm.at[:, pl.ds(base_of(0), _K)], aleb0)
        issue_gather(0)
        issue_idx(1, 1)

        def do_slot(ch, b):
            @pl.when(ch < n_i)
            def _():
                @pl.when(ch >= 2)
                def _():
                    pltpu.make_async_copy(mb[b], acc.at[ixb[b].at[1]],
                                          sems[b]).wait()

                wait_gather(b)

                @pl.when(ch + 1 < n_i)
                def _():
                    wait_idx(1 - b)
                    issue_gather(1 - b)

                gb = gbuf[b]
                ab = adbuf[b]
                alb = aleb[b]
                mbuf = mb[b]
                zero16 = jnp.zeros((16,), jnp.int32)
                for g in range(_K // 16):
                    ixb[b][1, pl.ds(g * 16, 16)] = eib[b][1, pl.ds(g * 16, 16)]
                NG = _K // 16

                @plsc.parallel_loop(0, NG * H, step=1, unroll=8)
                def _(i):
                    gv = i // H
                    hv = i % H
                    idx16 = iota + gv * 16
                    colh = jnp.full((16,), 128, jnp.int32) + hv
                    asrc = plsc.load_gather(gb, [idx16, colh])
                    adst = plsc.load_gather(ab, [idx16, zero16 + hv])
                    alev = plsc.load_gather(alb, [zero16 + hv, idx16])
                    a = asrc + adst + alev
                    a = jnp.where(a >= 0, a, 0.2 * a)
                    ex = jnp.exp(a)
                    plsc.store_scatter(mbuf, [idx16, colh], ex)
                    plsc.store_scatter(exb, [zero16 + (gv * H + hv), iota], ex)

                @plsc.parallel_loop(0, NG * 128, step=16, unroll=4)
                def _(i):
                    gv = i // 128
                    c0 = i % 128
                    hv = c0 // C
                    idx16 = iota + gv * 16
                    exv = plsc.load_gather(exb, [zero16 + (gv * H + hv), iota])
                    for u in range(16):
                        colv = zero16 + (c0 + u)
                        v = plsc.load_gather(gb, [idx16, colv]) * exv
                        plsc.store_scatter(mbuf, [idx16, colv], v)
                pltpu.async_copy(mbuf, acc.at[ixb[b].at[1]], sems[b], add=True)

                @pl.when(ch + 2 < n_i)
                def _():
                    issue_idx(ch + 2, b)

        def pair(j2, carry):
            do_slot(j2 * 2, 0)
            do_slot(j2 * 2 + 1, 1)
            return carry

        lax.fori_loop(0, (_MAXCPW + 1) // 2, pair, 0)
        for b in (0, 1):
            pltpu.make_async_copy(mb[b], acc.at[ixb[b].at[1]],
                                  sems[b]).wait()
        plsc.subcore_barrier()
        pltpu.sync_copy(acc.at[pl.ds(row0, _RPT)],
                        out_hbm.at[cid, pl.ds(row0, _RPT)])

    kern = pl.kernel(
        body,
        out_type=jax.ShapeDtypeStruct((2, _NPAD, _W), jnp.float32),
        mesh=mesh,
        compiler_params=pltpu.CompilerParams(use_tc_tiling_on_sc=False, needs_layout_passes=False),
        scratch_types=[
            pltpu.VMEM_SHARED((_NPAD, _W), jnp.float32),
            pltpu.VMEM((2, _K), jnp.int32),
            pltpu.VMEM((2, _K), jnp.int32),
            pltpu.VMEM((H, _K), jnp.float32),
            pltpu.VMEM((H, _K), jnp.float32),
            pltpu.VMEM((_K, 144), jnp.float32),
            pltpu.VMEM((_K, 144), jnp.float32),
            pltpu.VMEM((_K, 16), jnp.float32),
            pltpu.VMEM((_K, 16), jnp.float32),
            pltpu.VMEM((_K, _W), jnp.float32),
            pltpu.VMEM((_K, _W), jnp.float32),
            pltpu.VMEM((2, _K), jnp.int32),
            pltpu.VMEM((2, _K), jnp.int32),
            pltpu.VMEM(((_K // 16) * H, 16), jnp.float32),
            pltpu.SemaphoreType.DMA,
            pltpu.SemaphoreType.DMA,
            pltpu.SemaphoreType.DMA,
            pltpu.SemaphoreType.DMA,
            pltpu.SemaphoreType.DMA,
            pltpu.SemaphoreType.DMA,
        ],
    )
    return kern(tab, ald, ei, ale)


def _ale_tc(eaT, Wp1, Wp2, Wp3):
    BE = 2560

    def body(ea_ref, w1_ref, w2_ref, w3_ref, o1_ref, o2_ref, o3_ref):
        ea = ea_ref[...]
        dn = (((0,), (0,)), ((), ()))
        o1_ref[...] = lax.dot_general(w1_ref[...], ea, dn,
                                      preferred_element_type=jnp.float32)
        o2_ref[...] = lax.dot_general(w2_ref[...], ea, dn,
                                      preferred_element_type=jnp.float32)
        o3_ref[...] = lax.dot_general(w3_ref[...], ea, dn,
                                      preferred_element_type=jnp.float32)

    return pl.pallas_call(
        body,
        grid=(E // BE,),
        in_specs=[
            pl.BlockSpec((EDIM, BE), lambda i: (0, i)),
            pl.BlockSpec((EDIM, 8), lambda i: (0, 0)),
            pl.BlockSpec((EDIM, 8), lambda i: (0, 0)),
            pl.BlockSpec((EDIM, 8), lambda i: (0, 0)),
        ],
        out_specs=[
            pl.BlockSpec((8, BE), lambda i: (0, i)),
            pl.BlockSpec((8, BE), lambda i: (0, i)),
            pl.BlockSpec((8, BE), lambda i: (0, i)),
        ],
        out_shape=[jax.ShapeDtypeStruct((8, E), jnp.float32)] * 3,
    )(eaT, Wp1, Wp2, Wp3)


_BN = 2000


def _table_tc(xin, Wtab, Wald):
    def body(x_ref, wt_ref, wa_ref, tab_ref, ald_ref):
        xb = x_ref[...]
        tab_ref[...] = jnp.dot(xb, wt_ref[...],
                               preferred_element_type=jnp.float32)
        ald_ref[...] = jnp.dot(xb, wa_ref[...],
                               preferred_element_type=jnp.float32)

    return pl.pallas_call(
        body,
        grid=(N // _BN,),
        in_specs=[
            pl.BlockSpec((_BN, F_IN), lambda i: (i, 0)),
            pl.BlockSpec((F_IN, 144), lambda i: (0, 0)),
            pl.BlockSpec((F_IN, 16), lambda i: (0, 0)),
        ],
        out_specs=[
            pl.BlockSpec((_BN, 144), lambda i: (i, 0)),
            pl.BlockSpec((_BN, 16), lambda i: (i, 0)),
        ],
        out_shape=[jax.ShapeDtypeStruct((N, 144), jnp.float32),
                   jax.ShapeDtypeStruct((N, 16), jnp.float32)],
    )(xin, Wtab, Wald)


def _finalize_tc(acc, brow, Wtab, Wald):
    def body(a_ref, b_ref, wt_ref, wa_ref, tab_ref, ald_ref):
        a0 = a_ref[0]
        a1 = a_ref[1]
        num = a0[:, :128] + a1[:, :128]
        den = a0[:, 128:136] + a1[:, 128:136]
        r_i = lax.broadcasted_iota(jnp.int32, (8, 128), 0)
        r_j = lax.broadcasted_iota(jnp.int32, (8, 128), 1) // 16
        R = (r_i == r_j).astype(jnp.float32)
        den_rep = jnp.dot(den, R, preferred_element_type=jnp.float32)
        h = num / (den_rep + 1e-16) + b_ref[...]
        h = jnp.where(h > 0, h, jnp.exp(jnp.minimum(h, 0.0)) - 1.0)
        tab_ref[...] = jnp.dot(h, wt_ref[...],
                               preferred_element_type=jnp.float32)
        ald_ref[...] = jnp.dot(h, wa_ref[...],
                               preferred_element_type=jnp.float32)

    return pl.pallas_call(
        body,
        grid=(N // _BN,),
        in_specs=[
            pl.BlockSpec((2, _BN, _W), lambda i: (0, i, 0)),
            pl.BlockSpec((1, 128), lambda i: (0, 0)),
            pl.BlockSpec((F_IN, 144), lambda i: (0, 0)),
            pl.BlockSpec((F_IN, 16), lambda i: (0, 0)),
        ],
        out_specs=[
            pl.BlockSpec((_BN, 144), lambda i: (i, 0)),
            pl.BlockSpec((_BN, 16), lambda i: (i, 0)),
        ],
        out_shape=[jax.ShapeDtypeStruct((N, 144), jnp.float32),
                   jax.ShapeDtypeStruct((N, 16), jnp.float32)],
    )(acc, brow, Wtab, Wald)


def _final_tc(acc, brow):
    def body(a_ref, b_ref, o_ref):
        a0 = a_ref[0]
        a1 = a_ref[1]
        num = a0[:, :128] + a1[:, :128]
        den = a0[:, 128:129] + a1[:, 128:129]
        o_ref[...] = num / (den + 1e-16) + b_ref[...]

    return pl.pallas_call(
        body,
        grid=(N // _BN,),
        in_specs=[
            pl.BlockSpec((2, _BN, _W), lambda i: (0, i, 0)),
            pl.BlockSpec((1, 128), lambda i: (0, 0)),
        ],
        out_specs=pl.BlockSpec((_BN, 128), lambda i: (i, 0)),
        out_shape=jax.ShapeDtypeStruct((N, 128), jnp.float32),
    )(acc, brow)


def _mk_node_w(W, a_s, a_d, H, C):
    f32 = jnp.float32
    Wr = W.reshape(F_IN, H, C)
    Ws = jnp.einsum('fhc,hc->fh', Wr, a_s)
    Wd = jnp.einsum('fhc,hc->fh', Wr, a_d)
    Wtab = jnp.concatenate([W, Ws, jnp.zeros((F_IN, 16 - H), f32)], axis=1)
    Wald = jnp.concatenate([Wd, jnp.zeros((F_IN, 16 - H), f32)], axis=1)
    return Wtab, Wald


def _mk_edge_w(We, a_e, H, C):
    Wep = jnp.einsum('dhc,hc->dh', We.reshape(EDIM, H, C), a_e)
    return jnp.pad(Wep, ((0, 0), (0, 8 - H)))


def kernel(x, edge_index, edge_attr,
           W1, as1, ad1, We1, ae1, b1,
           W2, as2, ad2, We2, ae2, b2,
           W3, as3, ad3, We3, ae3, b3):
    eaT = edge_attr.T

    Wtab1, Wald1 = _mk_node_w(W1, as1, ad1, 8, 16)
    Wtab2, Wald2 = _mk_node_w(W2, as2, ad2, 8, 16)
    Wtab3, Wald3 = _mk_node_w(W3, as3, ad3, 1, 128)
    Wep1 = _mk_edge_w(We1, ae1, 8, 16)
    Wep2 = _mk_edge_w(We2, ae2, 8, 16)
    Wep3 = _mk_edge_w(We3, ae3, 1, 128)

    ale1, ale2, ale3 = _ale_tc(eaT, Wep1, Wep2, Wep3)

    tab1, ald1 = _table_tc(x, Wtab1, Wald1)
    acc1 = _sc_layer(tab1, ald1, edge_index, ale1, 8)
    tab2, ald2 = _finalize_tc(acc1, b1.reshape(1, 128), Wtab2, Wald2)
    acc2 = _sc_layer(tab2, ald2, edge_index, ale2, 8)
    tab3, ald3 = _finalize_tc(acc2, b2.reshape(1, 128), Wtab3, Wald3)
    acc3 = _sc_layer(tab3, ald3, edge_index, ale3[0:1], 1)
    return _final_tc(acc3, b3.reshape(1, 128))


# speedup vs baseline: 1.5879x; 1.5879x over previous
import functools

import jax
import jax.numpy as jnp
from jax import lax
from jax.experimental import pallas as pl
from jax.experimental.pallas import tpu as pltpu
from jax.experimental.pallas import tpu_sc as plsc

N = 10000
E = 320000
F_IN = 128
EDIM = 4

_K = 64
_NCHUNK = E // _K
_NW = 32
_MAXCPW = (_NCHUNK + _NW - 1) // _NW
_NPAD = 10112
_RPT = _NPAD // 16
_W = 136


def _sc_layer(tab, ald, ei, ale, H):
    C = 128 // H
    mesh = plsc.VectorSubcoreMesh(core_axis_name="c", subcore_axis_name="s")

    def body(tab_hbm, ald_hbm, ei_hbm, ale_hbm, out_hbm,
             acc, eib0, eib1, aleb0, aleb1, gbuf0, gbuf1, adbuf0, adbuf1,
             mb0, mb1, ixb0, ixb1, exb,
             semi0, semi1, semg0, semg1, sems0, sems1):
        cid = lax.axis_index("c")
        sid = lax.axis_index("s")
        wid = sid * 2 + cid
        eib = [eib0, eib1]
        aleb = [aleb0, aleb1]
        gbuf = [gbuf0, gbuf1]
        adbuf = [adbuf0, adbuf1]
        mb = [mb0, mb1]
        ixb = [ixb0, ixb1]
        semi = [semi0, semi1]
        semg = [semg0, semg1]
        sems = [sems0, sems1]

        zz = jnp.zeros((16,), jnp.float32)
        for mbuf in (mb0, mb1):
            for e in range(_K):
                for cg in range(8):
                    mbuf[e, pl.ds(cg * 16, 16)] = zz
                mbuf[e, pl.ds(_W - 16, 16)] = zz
        row0 = sid * _RPT
        nz = _RPT // _K
        for j in range(nz + 1):
            r = _K if j < nz else (_RPT - nz * _K)
            pltpu.sync_copy(mb0.at[pl.ds(0, r)],
                            acc.at[pl.ds(row0 + j * _K, r)])
        plsc.subcore_barrier()

        iota = lax.iota(jnp.int32, 16)
        n_i = (_NCHUNK - wid + _NW - 1) // _NW

        def base_of(ch):
            return (wid + ch * _NW) * _K

        def issue_idx(ch, b):
            pltpu.async_copy(ei_hbm.at[:, pl.ds(base_of(ch), _K)],
                             eib[b], semi[b])
            pltpu.async_copy(ale_hbm.at[:, pl.ds(base_of(ch), _K)],
                             aleb[b], semi[b])

        def wait_idx(b):
            pltpu.make_async_copy(ei_hbm.at[:, pl.ds(0, _K)],
                                  eib[b], semi[b]).wait()
            pltpu.make_async_copy(ale_hbm.at[:, pl.ds(0, _K)],
                                  aleb[b], semi[b]).wait()

        def issue_gather(b):
            pltpu.async_copy(tab_hbm.at[eib[b].at[0]], gbuf[b], semg[b])
            pltpu.async_copy(ald_hbm.at[eib[b].at[1]], adbuf[b], semg[b])

        def wait_gather(b):
            pltpu.make_async_copy(tab_hbm.at[eib[b].at[0]],
                                  gbuf[b], semg[b]).wait()
            pltpu.make_async_copy(ald_hbm.at[eib[b].at[1]],
                                  adbuf[b], semg[b]).wait()

        pltpu.sync_copy(ei_hbm.at[:, pl.ds(base_of(0), _K)], eib0)
        pltpu.sync_copy(ale_hbm.at[:, pl.ds(base_of(0), _K)], aleb0)
        issue_gather(0)
        issue_idx(1, 1)

        def do_slot(ch, b):
            @pl.when(ch < n_i)
            def _():
                @pl.when(ch >= 2)
                def _():
                    pltpu.make_async_copy(mb[b], acc.at[ixb[b].at[1]],
                                          sems[b]).wait()

                wait_gather(b)

                @pl.when(ch + 1 < n_i)
                def _():
                    wait_idx(1 - b)
                    issue_gather(1 - b)

                gb = gbuf[b]
                ab = adbuf[b]
                alb = aleb[b]
                mbuf = mb[b]
                zero16 = jnp.zeros((16,), jnp.int32)
                for g in range(_K // 16):
                    ixb[b][1, pl.ds(g * 16, 16)] = eib[b][1, pl.ds(g * 16, 16)]
                NG = _K // 16

                @plsc.parallel_loop(0, NG * H, step=1, unroll=8)
                def _(i):
                    gv = i // H
                    hv = i % H
                    idx16 = iota + gv * 16
                    colh = jnp.full((16,), 128, jnp.int32) + hv
                    asrc = plsc.load_gather(gb, [idx16, colh])
                    adst = plsc.load_gather(ab, [idx16, zero16 + hv])
                    alev = plsc.load_gather(alb, [zero16 + hv, idx16])
                    a = asrc + adst + alev
                    a = jnp.where(a >= 0, a, 0.2 * a)
                    ex = jnp.exp(a)
                    plsc.store_scatter(mbuf, [idx16, colh], ex)
                    plsc.store_scatter(exb, [zero16 + (gv * H + hv), iota], ex)

                @plsc.parallel_loop(0, NG * 128, step=1, unroll=16)
                def _(i):
                    gv = i // 128
                    c0 = i % 128
                    hv = c0 // C
                    idx16 = iota + gv * 16
                    exv = plsc.load_gather(exb, [zero16 + (gv * H + hv), iota])
                    colv = zero16 + c0
                    v = plsc.load_gather(gb, [idx16, colv]) * exv
                    plsc.store_scatter(mbuf, [idx16, colv], v)
                pltpu.async_copy(mbuf, acc.at[ixb[b].at[1]], sems[b], add=True)

                @pl.when(ch + 2 < n_i)
                def _():
                    issue_idx(ch + 2, b)

        def pair(j2, carry):
            do_slot(j2 * 2, 0)
            do_slot(j2 * 2 + 1, 1)
            return carry

        lax.fori_loop(0, (_MAXCPW + 1) // 2, pair, 0)
        for b in (0, 1):
            pltpu.make_async_copy(mb[b], acc.at[ixb[b].at[1]],
                                  sems[b]).wait()
        plsc.subcore_barrier()
        pltpu.sync_copy(acc.at[pl.ds(row0, _RPT)],
                        out_hbm.at[cid, pl.ds(row0, _RPT)])

    kern = pl.kernel(
        body,
        out_type=jax.ShapeDtypeStruct((2, _NPAD, _W), jnp.float32),
        mesh=mesh,
        compiler_params=pltpu.CompilerParams(use_tc_tiling_on_sc=False, needs_layout_passes=False),
        scratch_types=[
            pltpu.VMEM_SHARED((_NPAD, _W), jnp.float32),
            pltpu.VMEM((2, _K), jnp.int32),
            pltpu.VMEM((2, _K), jnp.int32),
            pltpu.VMEM((H, _K), jnp.float32),
            pltpu.VMEM((H, _K), jnp.float32),
            pltpu.VMEM((_K, 144), jnp.float32),
            pltpu.VMEM((_K, 144), jnp.float32),
            pltpu.VMEM((_K, 16), jnp.float32),
            pltpu.VMEM((_K, 16), jnp.float32),
            pltpu.VMEM((_K, _W), jnp.float32),
            pltpu.VMEM((_K, _W), jnp.float32),
            pltpu.VMEM((2, _K), jnp.int32),
            pltpu.VMEM((2, _K), jnp.int32),
            pltpu.VMEM(((_K // 16) * H, 16), jnp.float32),
            pltpu.SemaphoreType.DMA,
            pltpu.SemaphoreType.DMA,
            pltpu.SemaphoreType.DMA,
            pltpu.SemaphoreType.DMA,
            pltpu.SemaphoreType.DMA,
            pltpu.SemaphoreType.DMA,
        ],
    )
    return kern(tab, ald, ei, ale)


def _ale_tc(eaT, Wp1, Wp2, Wp3):
    BE = 2560

    def body(ea_ref, w1_ref, w2_ref, w3_ref, o1_ref, o2_ref, o3_ref):
        ea = ea_ref[...]
        dn = (((0,), (0,)), ((), ()))
        o1_ref[...] = lax.dot_general(w1_ref[...], ea, dn,
                                      preferred_element_type=jnp.float32)
        o2_ref[...] = lax.dot_general(w2_ref[...], ea, dn,
                                      preferred_element_type=jnp.float32)
        o3_ref[...] = lax.dot_general(w3_ref[...], ea, dn,
                                      preferred_element_type=jnp.float32)

    return pl.pallas_call(
        body,
        grid=(E // BE,),
        in_specs=[
            pl.BlockSpec((EDIM, BE), lambda i: (0, i)),
            pl.BlockSpec((EDIM, 8), lambda i: (0, 0)),
            pl.BlockSpec((EDIM, 8), lambda i: (0, 0)),
            pl.BlockSpec((EDIM, 8), lambda i: (0, 0)),
        ],
        out_specs=[
            pl.BlockSpec((8, BE), lambda i: (0, i)),
            pl.BlockSpec((8, BE), lambda i: (0, i)),
            pl.BlockSpec((8, BE), lambda i: (0, i)),
        ],
        out_shape=[jax.ShapeDtypeStruct((8, E), jnp.float32)] * 3,
    )(eaT, Wp1, Wp2, Wp3)


_BN = 2000


def _table_tc(xin, Wtab, Wald):
    def body(x_ref, wt_ref, wa_ref, tab_ref, ald_ref):
        xb = x_ref[...]
        tab_ref[...] = jnp.dot(xb, wt_ref[...],
                               preferred_element_type=jnp.float32)
        ald_ref[...] = jnp.dot(xb, wa_ref[...],
                               preferred_element_type=jnp.float32)

    return pl.pallas_call(
        body,
        grid=(N // _BN,),
        in_specs=[
            pl.BlockSpec((_BN, F_IN), lambda i: (i, 0)),
            pl.BlockSpec((F_IN, 144), lambda i: (0, 0)),
            pl.BlockSpec((F_IN, 16), lambda i: (0, 0)),
        ],
        out_specs=[
            pl.BlockSpec((_BN, 144), lambda i: (i, 0)),
            pl.BlockSpec((_BN, 16), lambda i: (i, 0)),
        ],
        out_shape=[jax.ShapeDtypeStruct((N, 144), jnp.float32),
                   jax.ShapeDtypeStruct((N, 16), jnp.float32)],
    )(xin, Wtab, Wald)


def _finalize_tc(acc, brow, Wtab, Wald):
    def body(a_ref, b_ref, wt_ref, wa_ref, tab_ref, ald_ref):
        a0 = a_ref[0]
        a1 = a_ref[1]
        num = a0[:, :128] + a1[:, :128]
        den = a0[:, 128:136] + a1[:, 128:136]
        r_i = lax.broadcasted_iota(jnp.int32, (8, 128), 0)
        r_j = lax.broadcasted_iota(jnp.int32, (8, 128), 1) // 16
        R = (r_i == r_j).astype(jnp.float32)
        den_rep = jnp.dot(den, R, preferred_element_type=jnp.float32)
        h = num / (den_rep + 1e-16) + b_ref[...]
        h = jnp.where(h > 0, h, jnp.exp(jnp.minimum(h, 0.0)) - 1.0)
        tab_ref[...] = jnp.dot(h, wt_ref[...],
                               preferred_element_type=jnp.float32)
        ald_ref[...] = jnp.dot(h, wa_ref[...],
                               preferred_element_type=jnp.float32)

    return pl.pallas_call(
        body,
        grid=(N // _BN,),
        in_specs=[
            pl.BlockSpec((2, _BN, _W), lambda i: (0, i, 0)),
            pl.BlockSpec((1, 128), lambda i: (0, 0)),
            pl.BlockSpec((F_IN, 144), lambda i: (0, 0)),
            pl.BlockSpec((F_IN, 16), lambda i: (0, 0)),
        ],
        out_specs=[
            pl.BlockSpec((_BN, 144), lambda i: (i, 0)),
            pl.BlockSpec((_BN, 16), lambda i: (i, 0)),
        ],
        out_shape=[jax.ShapeDtypeStruct((N, 144), jnp.float32),
                   jax.ShapeDtypeStruct((N, 16), jnp.float32)],
    )(acc, brow, Wtab, Wald)


def _final_tc(acc, brow):
    def body(a_ref, b_ref, o_ref):
        a0 = a_ref[0]
        a1 = a_ref[1]
        num = a0[:, :128] + a1[:, :128]
        den = a0[:, 128:129] + a1[:, 128:129]
        o_ref[...] = num / (den + 1e-16) + b_ref[...]

    return pl.pallas_call(
        body,
        grid=(N // _BN,),
        in_specs=[
            pl.BlockSpec((2, _BN, _W), lambda i: (0, i, 0)),
            pl.BlockSpec((1, 128), lambda i: (0, 0)),
        ],
        out_specs=pl.BlockSpec((_BN, 128), lambda i: (i, 0)),
        out_shape=jax.ShapeDtypeStruct((N, 128), jnp.float32),
    )(acc, brow)


def _mk_node_w(W, a_s, a_d, H, C):
    f32 = jnp.float32
    Wr = W.reshape(F_IN, H, C)
    Ws = jnp.einsum('fhc,hc->fh', Wr, a_s)
    Wd = jnp.einsum('fhc,hc->fh', Wr, a_d)
    Wtab = jnp.concatenate([W, Ws, jnp.zeros((F_IN, 16 - H), f32)], axis=1)
    Wald = jnp.concatenate([Wd, jnp.zeros((F_IN, 16 - H), f32)], axis=1)
    return Wtab, Wald


def _mk_edge_w(We, a_e, H, C):
    Wep = jnp.einsum('dhc,hc->dh', We.reshape(EDIM, H, C), a_e)
    return jnp.pad(Wep, ((0, 0), (0, 8 - H)))


def kernel(x, edge_index, edge_attr,
           W1, as1, ad1, We1, ae1, b1,
           W2, as2, ad2, We2, ae2, b2,
           W3, as3, ad3, We3, ae3, b3):
    eaT = edge_attr.T

    Wtab1, Wald1 = _mk_node_w(W1, as1, ad1, 8, 16)
    Wtab2, Wald2 = _mk_node_w(W2, as2, ad2, 8, 16)
    Wtab3, Wald3 = _mk_node_w(W3, as3, ad3, 1, 128)
    Wep1 = _mk_edge_w(We1, ae1, 8, 16)
    Wep2 = _mk_edge_w(We2, ae2, 8, 16)
    Wep3 = _mk_edge_w(We3, ae3, 1, 128)

    ale1, ale2, ale3 = _ale_tc(eaT, Wep1, Wep2, Wep3)

    tab1, ald1 = _table_tc(x, Wtab1, Wald1)
    acc1 = _sc_layer(tab1, ald1, edge_index, ale1, 8)
    tab2, ald2 = _finalize_tc(acc1, b1.reshape(1, 128), Wtab2, Wald2)
    acc2 = _sc_layer(tab2, ald2, edge_index, ale2, 8)
    tab3, ald3 = _finalize_tc(acc2, b2.reshape(1, 128), Wtab3, Wald3)
    acc3 = _sc_layer(tab3, ald3, edge_index, ale3[0:1], 1)
    return _final_tc(acc3, b3.reshape(1, 128))
